# baseline (speedup 1.0000x reference)
import jax
import jax.numpy as jnp
from jax import lax
from jax.experimental import pallas as pl
from jax.experimental.pallas import tpu as pltpu
from jax.experimental.pallas import tpu_sc as plsc

N = 10000
D = 128
E = 320000
NC = 2
NS = 16
NW = NC * NS
C = 128
EPW = 10240
EP = NW * EPW
NCHUNK = EPW // C
NCH = NCHUNK // 2
NBUF = 4
NGROUP = NCH // NBUF
NPAD = 12288
RPT = NPAD // NS
TPN = NPAD // NW
NPA = 10112
RPA = NPA // NS


def _mesh():
    return plsc.VectorSubcoreMesh(core_axis_name="c", subcore_axis_name="s")


def _deg_body(dst_hbm, degp_hbm, dsts_v, ones_v, z_v, dacc, dsem):
    c = lax.axis_index("c")
    s = lax.axis_index("s")
    wid = s * NC + c
    pltpu.sync_copy(dst_hbm.at[wid], dsts_v)
    for i in range(C // 16):
        ones_v[pl.ds(i * 16, 16)] = jnp.ones((16,), jnp.float32)
    for i in range(RPT // 16):
        z_v[pl.ds(i * 16, 16)] = jnp.zeros((16,), jnp.float32)
    pltpu.sync_copy(z_v, dacc.at[pl.ds(s * RPT, RPT)])
    plsc.subcore_barrier()

    def body(k, carry):
        pltpu.async_copy(ones_v, dacc.at[dsts_v.at[k]], dsem, add=True)
        return carry

    lax.fori_loop(0, NCHUNK, body, 0)
    pltpu.make_async_copy(dst_hbm.at[wid], dsts_v, dsem).wait()
    plsc.subcore_barrier()
    pltpu.sync_copy(dacc.at[pl.ds(s * RPT, RPT)],
                    degp_hbm.at[c].at[pl.ds(s * RPT, RPT)])


def _deg_partials(dst):
    return pl.kernel(
        _deg_body,
        out_type=jax.ShapeDtypeStruct((NC, NPAD), jnp.float32),
        mesh=_mesh(),
        scratch_types=[
            pltpu.VMEM((NCHUNK, C), jnp.int32),
            pltpu.VMEM((C,), jnp.float32),
            pltpu.VMEM((RPT,), jnp.float32),
            pltpu.VMEM_SHARED((NPAD,), jnp.float32),
            pltpu.SemaphoreType.DMA,
        ],
    )(dst)


def _rsqrt16(d):
    y = lax.bitcast_convert_type(
        jnp.int32(0x5F3759DF) - (lax.bitcast_convert_type(d, jnp.int32) >> 1),
        jnp.float32)
    for _ in range(3):
        y = y * (1.5 - 0.5 * d * y * y)
    return y


def _dis_body(degp_hbm, dis_hbm, a_v, b_v):
    c = lax.axis_index("c")
    s = lax.axis_index("s")
    wid = s * NC + c
    pltpu.sync_copy(degp_hbm.at[0].at[pl.ds(wid * TPN, TPN)], a_v)
    pltpu.sync_copy(degp_hbm.at[1].at[pl.ds(wid * TPN, TPN)], b_v)
    for i in range(TPN // 16):
        sl = pl.ds(i * 16, 16)
        a_v[sl] = _rsqrt16(a_v[sl] + b_v[sl] + 1.0)
    pltpu.sync_copy(a_v, dis_hbm.at[pl.ds(wid * TPN, TPN)])


def _dis_from_deg(degp):
    return pl.kernel(
        _dis_body,
        out_type=jax.ShapeDtypeStruct((NPAD,), jnp.float32),
        mesh=_mesh(),
        scratch_types=[
            pltpu.VMEM((TPN,), jnp.float32),
            pltpu.VMEM((TPN,), jnp.float32),
        ],
    )(degp)


def _agg_body(g_hbm, src_hbm, dst_hbm, p_hbm, srcs_v, dsts_v,
              si0, si1, di0, di1, r0, r1, acc, g0, g1):
    c = lax.axis_index("c")
    s = lax.axis_index("s")
    wid = s * NC + c
    pltpu.sync_copy(src_hbm.at[wid], srcs_v)
    pltpu.sync_copy(dst_hbm.at[wid], dsts_v)

    def zb(i, carry):
        for j in range(D // 16):
            r0[i, pl.ds(j * 16, 16)] = jnp.zeros((16,), jnp.float32)
        return carry

    lax.fori_loop(0, C, zb, 0)
    for k in range(RPA // C):
        pltpu.sync_copy(r0, acc.at[pl.ds(s * RPA + k * C, C)])
    rem = RPA % C
    pltpu.sync_copy(r0.at[pl.ds(0, rem)],
                    acc.at[pl.ds(s * RPA + RPA - rem, rem)])
    plsc.subcore_barrier()

    def idxrow(big, small, k):
        for j in range(C // 16):
            sl = pl.ds(j * 16, 16)
            small[sl] = big[k, sl]

    def group(j, carry):
        k = 2 * j
        idxrow(srcs_v, si0, k)
        idxrow(dsts_v, di0, k)
        idxrow(srcs_v, si1, k + 1)
        idxrow(dsts_v, di1, k + 1)
        cp0 = pltpu.async_copy(g_hbm.at[si0], r0, g0)
        cp1 = pltpu.async_copy(g_hbm.at[si1], r1, g1)
        cp0.wait()
        pltpu.sync_copy(r0, acc.at[di0], add=True)
        cp1.wait()
        pltpu.sync_copy(r1, acc.at[di1], add=True)
        return carry

    lax.fori_loop(0, NCH // 2, group, 0)
    plsc.subcore_barrier()
    pltpu.sync_copy(acc.at[pl.ds(s * RPA, RPA)],
                    p_hbm.at[c].at[pl.ds(s * RPA, RPA)])


def _edge_agg(g, src, dst):
    return pl.kernel(
        _agg_body,
        out_type=jax.ShapeDtypeStruct((NC, NPA, D), jnp.float32),
        mesh=_mesh(),
        scratch_types=(
            [pltpu.VMEM((NCH, C), jnp.int32)] * 2
            + [pltpu.VMEM((C,), jnp.int32)] * 4
            + [pltpu.VMEM((C, D), jnp.float32)] * 2
            + [pltpu.VMEM_SHARED((NPA, D), jnp.float32)]
            + [pltpu.SemaphoreType.DMA] * 2
        ),
    )(g, src, dst)


_BT = 1000


def _dot(a, b):
    return jnp.dot(a, b, preferred_element_type=jnp.float32,
                   precision=jax.lax.Precision.HIGHEST)


def _g_body(x_ref, w_ref, dis_ref, o_ref):
    o_ref[...] = dis_ref[...] * _dot(x_ref[...], w_ref[...])


def _tc_g(x, W, dis):
    return pl.pallas_call(
        _g_body,
        grid=(N // _BT,),
        in_specs=[
            pl.BlockSpec((_BT, D), lambda i: (i, 0)),
            pl.BlockSpec((D, D), lambda i: (0, 0)),
            pl.BlockSpec((_BT, 1), lambda i: (i, 0)),
        ],
        out_specs=pl.BlockSpec((_BT, D), lambda i: (i, 0)),
        out_shape=jax.ShapeDtypeStruct((N, D), jnp.float32),
    )(x, W, dis)


def _comb_body(pa_ref, pb_ref, g_ref, dis_ref, b_ref, w_ref, o_ref):
    agg = pa_ref[0] + pa_ref[1] + pb_ref[0] + pb_ref[1] + g_ref[...]
    z = jnp.maximum(dis_ref[...] * agg + b_ref[...], 0.0)
    o_ref[...] = dis_ref[...] * _dot(z, w_ref[...])


def _tc_combine_matmul(pa, pb, g, dis, b, W):
    return pl.pallas_call(
        _comb_body,
        grid=(N // _BT,),
        in_specs=[
            pl.BlockSpec((NC, _BT, D), lambda i: (0, i, 0)),
            pl.BlockSpec((NC, _BT, D), lambda i: (0, i, 0)),
            pl.BlockSpec((_BT, D), lambda i: (i, 0)),
            pl.BlockSpec((_BT, 1), lambda i: (i, 0)),
            pl.BlockSpec((1, D), lambda i: (0, 0)),
            pl.BlockSpec((D, D), lambda i: (0, 0)),
        ],
        out_specs=pl.BlockSpec((_BT, D), lambda i: (i, 0)),
        out_shape=jax.ShapeDtypeStruct((N, D), jnp.float32),
    )(pa, pb, g, dis, b, W)


def _fin_body(pa_ref, pb_ref, g_ref, dis_ref, b_ref, o_ref):
    agg = pa_ref[0] + pa_ref[1] + pb_ref[0] + pb_ref[1] + g_ref[...]
    o_ref[...] = jnp.maximum(dis_ref[...] * agg + b_ref[...], 0.0)


def _tc_final(pa, pb, g, dis, b):
    return pl.pallas_call(
        _fin_body,
        grid=(N // _BT,),
        in_specs=[
            pl.BlockSpec((NC, _BT, D), lambda i: (0, i, 0)),
            pl.BlockSpec((NC, _BT, D), lambda i: (0, i, 0)),
            pl.BlockSpec((_BT, D), lambda i: (i, 0)),
            pl.BlockSpec((_BT, 1), lambda i: (i, 0)),
            pl.BlockSpec((1, D), lambda i: (0, 0)),
        ],
        out_specs=pl.BlockSpec((_BT, D), lambda i: (i, 0)),
        out_shape=jax.ShapeDtypeStruct((N, D), jnp.float32),
    )(pa, pb, g, dis, b)


def kernel(x, edge_index, W1, b1, W2, b2):
    pad = EP - E
    src = jnp.concatenate(
        [edge_index[0].astype(jnp.int32),
         jnp.zeros((pad,), jnp.int32)]).reshape(NW, NCHUNK, C)
    dst = jnp.concatenate(
        [edge_index[1].astype(jnp.int32),
         jnp.full((pad,), N, jnp.int32)]).reshape(NW, NCHUNK, C)
    src_a, src_b = src[:, :NCH], src[:, NCH:]
    dst_a, dst_b = dst[:, :NCH], dst[:, NCH:]

    degp = _deg_partials(dst)
    dis_pad = _dis_from_deg(degp)
    dis = dis_pad[:N].reshape(N, 1)

    b1r = b1.reshape(1, D)
    b2r = b2.reshape(1, D)

    g1 = _tc_g(x, W1, dis)
    p1a = _edge_agg(g1, src_a, dst_a)
    p1b = _edge_agg(g1, src_b, dst_b)
    g2 = _tc_combine_matmul(p1a, p1b, g1, dis, b1r, W2)
    p2a = _edge_agg(g2, src_a, dst_a)
    p2b = _edge_agg(g2, src_b, dst_b)
    return _tc_final(p2a, p2b, g2, dis, b2r)


# speedup vs baseline: 1.1115x; 1.1115x over previous
import jax
import jax.numpy as jnp
from jax import lax
from jax.experimental import pallas as pl
from jax.experimental.pallas import tpu as pltpu
from jax.experimental.pallas import tpu_sc as plsc

N = 10000
D = 128
E = 320000
NC = 2
NS = 16
NW = NC * NS
C = 128
EPW = 10240
EP = NW * EPW
NCHUNK = EPW // C
NCH = NCHUNK
NBUF = 4
NGROUP = NCH // NBUF
NPAD = 12288
RPT = NPAD // NS
TPN = NPAD // NW
NPA = 10112
RPA = NPA // NS


def _mesh():
    return plsc.VectorSubcoreMesh(core_axis_name="c", subcore_axis_name="s")


def _deg_body(dst_hbm, degp_hbm, dsts_v, ones_v, z_v, dacc, dsem):
    c = lax.axis_index("c")
    s = lax.axis_index("s")
    wid = s * NC + c
    pltpu.sync_copy(dst_hbm.at[wid], dsts_v)
    for i in range(C // 16):
        ones_v[pl.ds(i * 16, 16)] = jnp.ones((16,), jnp.float32)
    for i in range(RPT // 16):
        z_v[pl.ds(i * 16, 16)] = jnp.zeros((16,), jnp.float32)
    pltpu.sync_copy(z_v, dacc.at[pl.ds(s * RPT, RPT)])
    plsc.subcore_barrier()

    def body(k, carry):
        pltpu.async_copy(ones_v, dacc.at[dsts_v.at[k]], dsem, add=True)
        return carry

    lax.fori_loop(0, NCHUNK, body, 0)
    pltpu.make_async_copy(dst_hbm.at[wid], dsts_v, dsem).wait()
    plsc.subcore_barrier()
    pltpu.sync_copy(dacc.at[pl.ds(s * RPT, RPT)],
                    degp_hbm.at[c].at[pl.ds(s * RPT, RPT)])


def _deg_partials(dst):
    return pl.kernel(
        _deg_body,
        out_type=jax.ShapeDtypeStruct((NC, NPAD), jnp.float32),
        mesh=_mesh(),
        scratch_types=[
            pltpu.VMEM((NCHUNK, C), jnp.int32),
            pltpu.VMEM((C,), jnp.float32),
            pltpu.VMEM((RPT,), jnp.float32),
            pltpu.VMEM_SHARED((NPAD,), jnp.float32),
            pltpu.SemaphoreType.DMA,
        ],
    )(dst)


def _rsqrt16(d):
    y = lax.bitcast_convert_type(
        jnp.int32(0x5F3759DF) - (lax.bitcast_convert_type(d, jnp.int32) >> 1),
        jnp.float32)
    for _ in range(3):
        y = y * (1.5 - 0.5 * d * y * y)
    return y


def _dis_body(degp_hbm, dis_hbm, a_v, b_v):
    c = lax.axis_index("c")
    s = lax.axis_index("s")
    wid = s * NC + c
    pltpu.sync_copy(degp_hbm.at[0].at[pl.ds(wid * TPN, TPN)], a_v)
    pltpu.sync_copy(degp_hbm.at[1].at[pl.ds(wid * TPN, TPN)], b_v)
    for i in range(TPN // 16):
        sl = pl.ds(i * 16, 16)
        a_v[sl] = _rsqrt16(a_v[sl] + b_v[sl] + 1.0)
    pltpu.sync_copy(a_v, dis_hbm.at[pl.ds(wid * TPN, TPN)])


def _dis_from_deg(degp):
    return pl.kernel(
        _dis_body,
        out_type=jax.ShapeDtypeStruct((NPAD,), jnp.float32),
        mesh=_mesh(),
        scratch_types=[
            pltpu.VMEM((TPN,), jnp.float32),
            pltpu.VMEM((TPN,), jnp.float32),
        ],
    )(degp)


def _agg_body(g_hbm, src_hbm, dst_hbm, p_hbm, srcs_v, dsts_v,
              si0, si1, di0, di1, r0, r1, acc, g0, g1):
    c = lax.axis_index("c")
    s = lax.axis_index("s")
    wid = s * NC + c
    pltpu.sync_copy(src_hbm.at[wid], srcs_v)
    pltpu.sync_copy(dst_hbm.at[wid], dsts_v)

    def zb(i, carry):
        for j in range(D // 16):
            r0[i, pl.ds(j * 16, 16)] = jnp.zeros((16,), jnp.float32)
        return carry

    lax.fori_loop(0, C, zb, 0)
    for k in range(RPA // C):
        pltpu.sync_copy(r0, acc.at[pl.ds(s * RPA + k * C, C)])
    rem = RPA % C
    pltpu.sync_copy(r0.at[pl.ds(0, rem)],
                    acc.at[pl.ds(s * RPA + RPA - rem, rem)])
    plsc.subcore_barrier()

    def idxrow(big, small, k):
        for j in range(C // 16):
            sl = pl.ds(j * 16, 16)
            small[sl] = big[k, sl]

    def chunk(k, carry):
        idxrow(srcs_v, si0, k)
        idxrow(dsts_v, di0, k)
        pltpu.async_copy(g_hbm.at[si0], r0, g0).wait()
        pltpu.sync_copy(r0, acc.at[di0], add=True)
        return carry

    lax.fori_loop(0, NCH, chunk, 0)
    plsc.subcore_barrier()
    pltpu.sync_copy(acc.at[pl.ds(s * RPA, RPA)],
                    p_hbm.at[c].at[pl.ds(s * RPA, RPA)])


def _edge_agg(g, src, dst):
    return pl.kernel(
        _agg_body,
        out_type=jax.ShapeDtypeStruct((NC, NPA, D), jnp.float32),
        mesh=_mesh(),
        scratch_types=(
            [pltpu.VMEM((NCH, C), jnp.int32)] * 2
            + [pltpu.VMEM((C,), jnp.int32)] * 4
            + [pltpu.VMEM((C, D), jnp.float32)] * 2
            + [pltpu.VMEM_SHARED((NPA, D), jnp.float32)]
            + [pltpu.SemaphoreType.DMA] * 2
        ),
    )(g, src, dst)


_BT = 1000


def _dot(a, b):
    return jnp.dot(a, b, preferred_element_type=jnp.float32,
                   precision=jax.lax.Precision.HIGHEST)


def _g_body(x_ref, w_ref, dis_ref, o_ref):
    o_ref[...] = dis_ref[...] * _dot(x_ref[...], w_ref[...])


def _tc_g(x, W, dis):
    return pl.pallas_call(
        _g_body,
        grid=(N // _BT,),
        in_specs=[
            pl.BlockSpec((_BT, D), lambda i: (i, 0)),
            pl.BlockSpec((D, D), lambda i: (0, 0)),
            pl.BlockSpec((_BT, 1), lambda i: (i, 0)),
        ],
        out_specs=pl.BlockSpec((_BT, D), lambda i: (i, 0)),
        out_shape=jax.ShapeDtypeStruct((N, D), jnp.float32),
    )(x, W, dis)


def _comb_body(pa_ref, g_ref, dis_ref, b_ref, w_ref, o_ref):
    agg = pa_ref[0] + pa_ref[1] + g_ref[...]
    z = jnp.maximum(dis_ref[...] * agg + b_ref[...], 0.0)
    o_ref[...] = dis_ref[...] * _dot(z, w_ref[...])


def _tc_combine_matmul(pa, g, dis, b, W):
    return pl.pallas_call(
        _comb_body,
        grid=(N // _BT,),
        in_specs=[
            pl.BlockSpec((NC, _BT, D), lambda i: (0, i, 0)),
            pl.BlockSpec((_BT, D), lambda i: (i, 0)),
            pl.BlockSpec((_BT, 1), lambda i: (i, 0)),
            pl.BlockSpec((1, D), lambda i: (0, 0)),
            pl.BlockSpec((D, D), lambda i: (0, 0)),
        ],
        out_specs=pl.BlockSpec((_BT, D), lambda i: (i, 0)),
        out_shape=jax.ShapeDtypeStruct((N, D), jnp.float32),
    )(pa, g, dis, b, W)


def _fin_body(pa_ref, g_ref, dis_ref, b_ref, o_ref):
    agg = pa_ref[0] + pa_ref[1] + g_ref[...]
    o_ref[...] = jnp.maximum(dis_ref[...] * agg + b_ref[...], 0.0)


def _tc_final(pa, g, dis, b):
    return pl.pallas_call(
        _fin_body,
        grid=(N // _BT,),
        in_specs=[
            pl.BlockSpec((NC, _BT, D), lambda i: (0, i, 0)),
            pl.BlockSpec((_BT, D), lambda i: (i, 0)),
            pl.BlockSpec((_BT, 1), lambda i: (i, 0)),
            pl.BlockSpec((1, D), lambda i: (0, 0)),
        ],
        out_specs=pl.BlockSpec((_BT, D), lambda i: (i, 0)),
        out_shape=jax.ShapeDtypeStruct((N, D), jnp.float32),
    )(pa, g, dis, b)


def kernel(x, edge_index, W1, b1, W2, b2):
    pad = EP - E
    src = jnp.concatenate(
        [edge_index[0].astype(jnp.int32),
         jnp.zeros((pad,), jnp.int32)]).reshape(NW, NCHUNK, C)
    dst = jnp.concatenate(
        [edge_index[1].astype(jnp.int32),
         jnp.full((pad,), N, jnp.int32)]).reshape(NW, NCHUNK, C)

    degp = _deg_partials(dst)
    dis_pad = _dis_from_deg(degp)
    dis = dis_pad[:N].reshape(N, 1)

    b1r = b1.reshape(1, D)
    b2r = b2.reshape(1, D)

    g1 = _tc_g(x, W1, dis)
    p1 = _edge_agg(g1, src, dst)
    g2 = _tc_combine_matmul(p1, g1, dis, b1r, W2)
    p2 = _edge_agg(g2, src, dst)
    return _tc_final(p2, g2, dis, b2r)


# speedup vs baseline: 1.3047x; 1.1739x over previous
import jax
import jax.numpy as jnp
from jax import lax
from jax.experimental import pallas as pl
from jax.experimental.pallas import tpu as pltpu
from jax.experimental.pallas import tpu_sc as plsc

N = 10000
D = 128
E = 320000
NC = 2
NS = 16
NW = NC * NS
C = 128
EPW = 10112
EP = NW * EPW
NCHUNK = EPW // C
NCH = NCHUNK
NBUF = 4
NGROUP = NCH // NBUF
NPAD = 12288
RPT = NPAD // NS
TPN = NPAD // NW
NPA = NPAD
RPA = NPA // NS


def _mesh():
    return plsc.VectorSubcoreMesh(core_axis_name="c", subcore_axis_name="s")


def _deg_body(dst_hbm, degp_hbm, dsts_v, ones_v, z_v, dacc, dsem):
    c = lax.axis_index("c")
    s = lax.axis_index("s")
    wid = s * NC + c
    pltpu.sync_copy(dst_hbm.at[wid], dsts_v)
    for i in range(C // 16):
        ones_v[pl.ds(i * 16, 16)] = jnp.ones((16,), jnp.float32)
    for i in range(RPT // 16):
        z_v[pl.ds(i * 16, 16)] = jnp.zeros((16,), jnp.float32)
    pltpu.sync_copy(z_v, dacc.at[pl.ds(s * RPT, RPT)])
    plsc.subcore_barrier()

    def body(k, carry):
        pltpu.async_copy(ones_v, dacc.at[dsts_v.at[k]], dsem, add=True)
        return carry

    lax.fori_loop(0, NCHUNK, body, 0)
    pltpu.make_async_copy(dst_hbm.at[wid], dsts_v, dsem).wait()
    plsc.subcore_barrier()
    pltpu.sync_copy(dacc.at[pl.ds(s * RPT, RPT)],
                    degp_hbm.at[c].at[pl.ds(s * RPT, RPT)])


def _deg_partials(dst):
    return pl.kernel(
        _deg_body,
        out_type=jax.ShapeDtypeStruct((NC, NPAD), jnp.float32),
        mesh=_mesh(),
        scratch_types=[
            pltpu.VMEM((NCHUNK, C), jnp.int32),
            pltpu.VMEM((C,), jnp.float32),
            pltpu.VMEM((RPT,), jnp.float32),
            pltpu.VMEM_SHARED((NPAD,), jnp.float32),
            pltpu.SemaphoreType.DMA,
        ],
    )(dst)


def _rsqrt16(d):
    y = lax.bitcast_convert_type(
        jnp.int32(0x5F3759DF) - (lax.bitcast_convert_type(d, jnp.int32) >> 1),
        jnp.float32)
    for _ in range(3):
        y = y * (1.5 - 0.5 * d * y * y)
    return y


def _dis_body(degp_hbm, dis_hbm, a_v, b_v):
    c = lax.axis_index("c")
    s = lax.axis_index("s")
    wid = s * NC + c
    pltpu.sync_copy(degp_hbm.at[0].at[pl.ds(wid * TPN, TPN)], a_v)
    pltpu.sync_copy(degp_hbm.at[1].at[pl.ds(wid * TPN, TPN)], b_v)
    for i in range(TPN // 16):
        sl = pl.ds(i * 16, 16)
        a_v[sl] = _rsqrt16(a_v[sl] + b_v[sl] + 1.0)
    pltpu.sync_copy(a_v, dis_hbm.at[pl.ds(wid * TPN, TPN)])


def _dis_from_deg(degp):
    return pl.kernel(
        _dis_body,
        out_type=jax.ShapeDtypeStruct((NPAD,), jnp.float32),
        mesh=_mesh(),
        scratch_types=[
            pltpu.VMEM((TPN,), jnp.float32),
            pltpu.VMEM((TPN,), jnp.float32),
        ],
    )(degp)


def _agg_body(g_hbm, src_hbm, dst_hbm, p_hbm, si0, di0, r0, acc, g0):
    c = lax.axis_index("c")
    s = lax.axis_index("s")
    wid = s * NC + c

    def zb(i, carry):
        for j in range(D // 16):
            r0[i, pl.ds(j * 16, 16)] = jnp.zeros((16,), jnp.float32)
        return carry

    lax.fori_loop(0, C, zb, 0)
    for k in range(RPA // C):
        pltpu.sync_copy(r0, acc.at[pl.ds(s * RPA + k * C, C)])
    plsc.subcore_barrier()

    def chunk(k, carry):
        pltpu.sync_copy(src_hbm.at[wid].at[k], si0)
        pltpu.sync_copy(dst_hbm.at[wid].at[k], di0)
        pltpu.async_copy(g_hbm.at[si0], r0, g0).wait()
        pltpu.sync_copy(r0, acc.at[di0], add=True)
        return carry

    lax.fori_loop(0, NCH, chunk, 0)
    plsc.subcore_barrier()
    pltpu.sync_copy(acc.at[pl.ds(s * RPA, RPA)],
                    p_hbm.at[c].at[pl.ds(s * RPA, RPA)])


def _edge_agg(g, src, dst):
    return pl.kernel(
        _agg_body,
        out_type=jax.ShapeDtypeStruct((NC, NPA, D), jnp.float32),
        mesh=_mesh(),
        scratch_types=(
            [pltpu.VMEM((C,), jnp.int32)] * 2
            + [pltpu.VMEM((C, D), jnp.float32)]
            + [pltpu.VMEM_SHARED((NPA, D), jnp.float32)]
            + [pltpu.SemaphoreType.DMA]
        ),
    )(g, src, dst)


_BT = 1000


def _dot(a, b):
    return jnp.dot(a, b, preferred_element_type=jnp.float32,
                   precision=jax.lax.Precision.HIGHEST)


def _g_body(x_ref, w_ref, dis_ref, o_ref):
    o_ref[...] = dis_ref[...] * _dot(x_ref[...], w_ref[...])


def _tc_g(x, W, dis):
    return pl.pallas_call(
        _g_body,
        grid=(N // _BT,),
        in_specs=[
            pl.BlockSpec((_BT, D), lambda i: (i, 0)),
            pl.BlockSpec((D, D), lambda i: (0, 0)),
            pl.BlockSpec((_BT, 1), lambda i: (i, 0)),
        ],
        out_specs=pl.BlockSpec((_BT, D), lambda i: (i, 0)),
        out_shape=jax.ShapeDtypeStruct((N, D), jnp.float32),
    )(x, W, dis)


def _comb_body(pa_ref, g_ref, dis_ref, b_ref, w_ref, o_ref):
    agg = pa_ref[0] + pa_ref[1] + g_ref[...]
    z = jnp.maximum(dis_ref[...] * agg + b_ref[...], 0.0)
    o_ref[...] = dis_ref[...] * _dot(z, w_ref[...])


def _tc_combine_matmul(pa, g, dis, b, W):
    return pl.pallas_call(
        _comb_body,
        grid=(N // _BT,),
        in_specs=[
            pl.BlockSpec((NC, _BT, D), lambda i: (0, i, 0)),
            pl.BlockSpec((_BT, D), lambda i: (i, 0)),
            pl.BlockSpec((_BT, 1), lambda i: (i, 0)),
            pl.BlockSpec((1, D), lambda i: (0, 0)),
            pl.BlockSpec((D, D), lambda i: (0, 0)),
        ],
        out_specs=pl.BlockSpec((_BT, D), lambda i: (i, 0)),
        out_shape=jax.ShapeDtypeStruct((N, D), jnp.float32),
    )(pa, g, dis, b, W)


def _fin_body(pa_ref, g_ref, dis_ref, b_ref, o_ref):
    agg = pa_ref[0] + pa_ref[1] + g_ref[...]
    o_ref[...] = jnp.maximum(dis_ref[...] * agg + b_ref[...], 0.0)


def _tc_final(pa, g, dis, b):
    return pl.pallas_call(
        _fin_body,
        grid=(N // _BT,),
        in_specs=[
            pl.BlockSpec((NC, _BT, D), lambda i: (0, i, 0)),
            pl.BlockSpec((_BT, D), lambda i: (i, 0)),
            pl.BlockSpec((_BT, 1), lambda i: (i, 0)),
            pl.BlockSpec((1, D), lambda i: (0, 0)),
        ],
        out_specs=pl.BlockSpec((_BT, D), lambda i: (i, 0)),
        out_shape=jax.ShapeDtypeStruct((N, D), jnp.float32),
    )(pa, g, dis, b)


def kernel(x, edge_index, W1, b1, W2, b2):
    pad = EP - E
    src = jnp.concatenate(
        [edge_index[0].astype(jnp.int32),
         jnp.zeros((pad,), jnp.int32)]).reshape(NW, NCHUNK, C)
    dst = jnp.concatenate(
        [edge_index[1].astype(jnp.int32),
         jnp.full((pad,), N, jnp.int32)]).reshape(NW, NCHUNK, C)

    degp = _deg_partials(dst)
    dis_pad = _dis_from_deg(degp)
    dis = dis_pad[:N].reshape(N, 1)

    b1r = b1.reshape(1, D)
    b2r = b2.reshape(1, D)

    g1 = _tc_g(x, W1, dis)
    p1 = _edge_agg(g1, src, dst)
    g2 = _tc_combine_matmul(p1, g1, dis, b1r, W2)
    p2 = _edge_agg(g2, src, dst)
    return _tc_final(p2, g2, dis, b2r)
